# baseline (speedup 1.0000x reference)
import functools

import jax
import jax.numpy as jnp
from jax import lax
from jax.experimental import pallas as pl
from jax.experimental.pallas import tpu as pltpu
from jax.experimental.pallas import tpu_sc as plsc

K = 8100
D = 256
N_TOK = 8192
TM = 512
N_TILES = N_TOK // TM
KCHUNK = 2704
EPS = 1e-12

_SC_NC = 2
_SC_NS = 16
_NW = _SC_NC * _SC_NS
_BPW = N_TOK // _NW


NWIN = -(-K // KCHUNK)
WPAD = 2816
NGRP = WPAD // 128


def _norm_body(w_ref, wn_ref, wb_ref):
    w = w_ref[...]
    n = jnp.sqrt(jnp.sum(w * w, axis=1, keepdims=True))
    wn = w / jnp.maximum(n, EPS)
    wn_ref[...] = wn
    m2 = (-2.0 * wn).astype(jnp.bfloat16)
    parts = []
    for c0 in range(0, K, KCHUNK):
        nc = min(KCHUNK, K - c0)
        parts.append(m2[c0:c0 + nc, :])
        parts.append(jnp.zeros((WPAD - nc, D), jnp.bfloat16))
    wb_ref[...] = jnp.concatenate(parts, axis=0)


_norm_call = pl.pallas_call(
    _norm_body,
    out_shape=[
        jax.ShapeDtypeStruct((K, D), jnp.float32),
        jax.ShapeDtypeStruct((NWIN * WPAD, D), jnp.bfloat16),
    ],
)


def _tc_body(z_ref, wb_ref, idx_ref):
    z = z_ref[...]
    zn = z / jnp.maximum(jnp.sqrt(jnp.sum(z * z, axis=1, keepdims=True)), EPS)
    zb = zn.astype(jnp.bfloat16)
    lane = lax.broadcasted_iota(jnp.int32, (TM, 128), 1)
    runv = jnp.full((TM,), jnp.inf, jnp.float32)
    runi = jnp.zeros((TM,), jnp.int32)
    for w in range(NWIN):
        c0 = w * KCHUNK
        nc = min(KCHUNK, K - c0)
        s2 = lax.dot_general(zb, wb_ref[w * WPAD:(w + 1) * WPAD, :],
                             (((1,), (1,)), ((), ())),
                             preferred_element_type=jnp.float32)
        d = 2.0 + s2
        groups = [lax.slice(d, (0, 128 * j), (TM, 128 * (j + 1)))
                  for j in range(NGRP)]
        tail = jnp.where(128 * (NGRP - 1) + lane >= nc, jnp.inf, groups[-1])
        groups[-1] = tail
        runmin = groups[0]
        runj = jnp.zeros((TM, 128), jnp.int32)
        for j in range(1, NGRP):
            m = groups[j] < runmin
            runmin = jnp.where(m, groups[j], runmin)
            runj = jnp.where(m, jnp.int32(j), runj)
        vc = jnp.min(runmin, axis=1)
        cand = jnp.where(runmin == vc[:, None], runj * 128 + lane,
                         jnp.int32(1 << 28))
        ic = c0 + jnp.min(cand, axis=1)
        take = vc < runv
        runi = jnp.where(take, ic, runi)
        runv = jnp.where(take, vc, runv).astype(jnp.bfloat16).astype(jnp.float32)
    idx_ref[0, 0, :] = runi


HALF = N_TOK // 2
HTILES = HALF // TM
_BPW_H = HALF // _NW

_tc_call = pl.pallas_call(
    _tc_body,
    grid=(HTILES,),
    in_specs=[
        pl.BlockSpec((TM, D), lambda i: (i, 0)),
        pl.BlockSpec((NWIN * WPAD, D), lambda i: (0, 0)),
    ],
    out_specs=pl.BlockSpec((1, 1, TM), lambda i: (i, 0, 0)),
    out_shape=jax.ShapeDtypeStruct((HTILES, 1, TM), jnp.int32),
)


def _sc_gather_body(table_hbm, idx_hbm, out_hbm, idx_v, rows_v, sem):
    wid = lax.axis_index("s") * _SC_NC + lax.axis_index("c")
    base = wid * _BPW_H
    pltpu.sync_copy(idx_hbm.at[pl.ds(base, _BPW_H)], idx_v)
    pltpu.async_copy(table_hbm.at[idx_v], rows_v, sem).wait()
    pltpu.sync_copy(rows_v, out_hbm.at[pl.ds(base, _BPW_H)])


@functools.cache
def _sc_gather():
    return functools.partial(
        pl.kernel,
        mesh=plsc.VectorSubcoreMesh(core_axis_name="c", subcore_axis_name="s"),
        out_type=jax.ShapeDtypeStruct((HALF, D), jnp.float32),
        scratch_types=[
            pltpu.VMEM((_BPW_H,), jnp.int32),
            pltpu.VMEM((_BPW_H, D), jnp.float32),
            pltpu.SemaphoreType.DMA,
        ],
    )(_sc_gather_body)


def kernel(z_e, weight):
    b, d_, h, w_ = z_e.shape
    z = jnp.transpose(z_e, (0, 2, 3, 1)).reshape(-1, d_)
    wn, wb = _norm_call(weight)
    f0 = _tc_call(z[:HALF], wb).reshape(-1)
    q0 = _sc_gather()(wn, f0)
    f1 = _tc_call(z[HALF:], wb).reshape(-1)
    q1 = _sc_gather()(wn, f1)
    zq_flat = jnp.concatenate([q0, q1], axis=0)
    idx_flat = jnp.concatenate([f0, f1])
    z_q = jnp.transpose(zq_flat.reshape(b, h, w_, d_), (0, 3, 1, 2))
    return z_q, idx_flat.reshape(b, h, w_)


# speedup vs baseline: 1.1783x; 1.1783x over previous
import functools

import jax
import jax.numpy as jnp
from jax import lax
from jax.experimental import pallas as pl
from jax.experimental.pallas import tpu as pltpu
from jax.experimental.pallas import tpu_sc as plsc

K = 8100
D = 256
N_TOK = 8192
TM = 512
N_TILES = N_TOK // TM
KCHUNK = 2704
EPS = 1e-12

_SC_NC = 2
_SC_NS = 16
_NW = _SC_NC * _SC_NS
_BPW = N_TOK // _NW


NWIN = -(-K // KCHUNK)
WPAD = 2816
NGRP = WPAD // 128


def _norm_body(w_ref, wn_ref, wb_ref):
    w = w_ref[...]
    n = jnp.sqrt(jnp.sum(w * w, axis=1, keepdims=True))
    wn = w / jnp.maximum(n, EPS)
    wn_ref[...] = wn
    m2 = (-2.0 * wn).astype(jnp.bfloat16)
    parts = []
    for c0 in range(0, K, KCHUNK):
        nc = min(KCHUNK, K - c0)
        parts.append(m2[c0:c0 + nc, :])
        parts.append(jnp.zeros((WPAD - nc, D), jnp.bfloat16))
    wb_ref[...] = jnp.concatenate(parts, axis=0)


_norm_call = pl.pallas_call(
    _norm_body,
    out_shape=[
        jax.ShapeDtypeStruct((K, D), jnp.float32),
        jax.ShapeDtypeStruct((NWIN * WPAD, D), jnp.bfloat16),
    ],
)


def _tc_body(z_ref, wb_ref, idx_ref):
    z = z_ref[...]
    zn = z / jnp.maximum(jnp.sqrt(jnp.sum(z * z, axis=1, keepdims=True)), EPS)
    zb = zn.astype(jnp.bfloat16)
    lane = lax.broadcasted_iota(jnp.int32, (TM, 128), 1)
    runv = jnp.full((TM,), jnp.inf, jnp.float32)
    runi = jnp.zeros((TM,), jnp.int32)
    for w in range(NWIN):
        c0 = w * KCHUNK
        nc = min(KCHUNK, K - c0)
        s2 = lax.dot_general(zb, wb_ref[w * WPAD:(w + 1) * WPAD, :],
                             (((1,), (1,)), ((), ())),
                             preferred_element_type=jnp.float32)
        d = 2.0 + s2
        groups = [lax.slice(d, (0, 128 * j), (TM, 128 * (j + 1)))
                  for j in range(NGRP)]
        tail = jnp.where(128 * (NGRP - 1) + lane >= nc, jnp.inf, groups[-1])
        groups[-1] = tail
        runmin = groups[0]
        runj = jnp.zeros((TM, 128), jnp.int32)
        for j in range(1, NGRP):
            m = groups[j] < runmin
            runmin = jnp.where(m, groups[j], runmin)
            runj = jnp.where(m, jnp.int32(j), runj)
        vc = jnp.min(runmin, axis=1)
        cand = jnp.where(runmin == vc[:, None], runj * 128 + lane,
                         jnp.int32(1 << 28))
        ic = c0 + jnp.min(cand, axis=1)
        take = vc < runv
        runi = jnp.where(take, ic, runi)
        runv = jnp.where(take, vc, runv).astype(jnp.bfloat16).astype(jnp.float32)
    idx_ref[0, 0, :] = runi


_tc_call = pl.pallas_call(
    _tc_body,
    grid=(N_TILES,),
    in_specs=[
        pl.BlockSpec((TM, D), lambda i: (i, 0)),
        pl.BlockSpec((NWIN * WPAD, D), lambda i: (0, 0)),
    ],
    out_specs=pl.BlockSpec((1, 1, TM), lambda i: (i, 0, 0)),
    out_shape=jax.ShapeDtypeStruct((N_TILES, 1, TM), jnp.int32),
)


def _sc_gather_body(table_hbm, idx_hbm, out_hbm, idx_v, rows_v, sem):
    wid = lax.axis_index("s") * _SC_NC + lax.axis_index("c")
    base = wid * _BPW
    pltpu.sync_copy(idx_hbm.at[pl.ds(base, _BPW)], idx_v)
    pltpu.async_copy(table_hbm.at[idx_v], rows_v, sem).wait()
    pltpu.sync_copy(rows_v, out_hbm.at[pl.ds(base, _BPW)])


@functools.cache
def _sc_gather():
    return functools.partial(
        pl.kernel,
        mesh=plsc.VectorSubcoreMesh(core_axis_name="c", subcore_axis_name="s"),
        out_type=jax.ShapeDtypeStruct((N_TOK, D), jnp.float32),
        scratch_types=[
            pltpu.VMEM((_BPW,), jnp.int32),
            pltpu.VMEM((_BPW, D), jnp.float32),
            pltpu.SemaphoreType.DMA,
        ],
    )(_sc_gather_body)


def kernel(z_e, weight):
    b, d_, h, w_ = z_e.shape
    z = jnp.transpose(z_e, (0, 2, 3, 1)).reshape(-1, d_)
    wn, wb = _norm_call(weight)
    idx3 = _tc_call(z, wb)
    idx_flat = idx3.reshape(-1)
    zq_flat = _sc_gather()(wn, idx_flat)
    z_q = jnp.transpose(zq_flat.reshape(b, h, w_, d_), (0, 3, 1, 2))
    return z_q, idx_flat.reshape(b, h, w_)
